# baseline (speedup 1.0000x reference)
import jax
import jax.numpy as jnp
from jax.experimental import pallas as pl
from jax.experimental.pallas import tpu as pltpu

NUM_EXPERTS = 16
HID = 1024
B_, C_, T_ = 2, 1024, 4096
LANES = 128
TH = T_ // LANES
RTOT = B_ * C_
RB = 256
NT = RTOT // RB
NB0 = C_ // RB


def _body(x_ref, gw_ref, gb_ref, gam_ref, bet_ref, out_ref,
          stash_ref, pbacc_ref, gsel_ref, bsel_ref):
    p = pl.program_id(0)
    j = pl.program_id(1)

    @pl.when(p == 0)
    def _reduce():
        xb = x_ref[...]
        stash_ref[pl.ds(j * RB, RB)] = xb
        s = xb[:, 0:8, :]
        for k in range(1, TH // 8):
            s = s + xb[:, 8 * k:8 * (k + 1), :]
        partial = jnp.sum(s, axis=1)
        cbase = (j % NB0) * RB
        pbj = jax.lax.dot_general(
            gw_ref[:, pl.ds(cbase, RB)], partial,
            (((1,), (0,)), ((), ())),
            preferred_element_type=jnp.float32,
        )
        rbase = jnp.where(j < NB0, 0, NUM_EXPERTS)

        @pl.when(j % NB0 == 0)
        def _():
            pbacc_ref[pl.ds(rbase, NUM_EXPERTS)] = pbj

        @pl.when(j % NB0 != 0)
        def _():
            pbacc_ref[pl.ds(rbase, NUM_EXPERTS)] += pbj

        @pl.when(j == NT - 1)
        def _gate():
            iota = jax.lax.broadcasted_iota(
                jnp.int32, (NUM_EXPERTS, 1), 0)
            for b in range(B_):
                pb = pbacc_ref[b * NUM_EXPERTS:(b + 1) * NUM_EXPERTS, :]
                scores = (jnp.sum(pb, axis=-1, keepdims=True) * (1.0 / T_)
                          + gb_ref[...])
                m = jnp.max(scores, axis=0, keepdims=True)
                idx = jnp.min(
                    jnp.where(scores >= m, iota, NUM_EXPERTS),
                    axis=0, keepdims=True)
                sel = iota == idx
                gsel_ref[b:b + 1, :] = jnp.sum(
                    jnp.where(sel, gam_ref[...], 0.0), axis=0, keepdims=True)
                bsel_ref[b:b + 1, :] = jnp.sum(
                    jnp.where(sel, bet_ref[...], 0.0), axis=0, keepdims=True)

    @pl.when(p == 1)
    def _apply():
        gsel = gsel_ref[...]
        bsel = bsel_ref[...]
        g = jnp.where(j < NB0, gsel[0:1, 0:1], gsel[1:2, 0:1])
        b = jnp.where(j < NB0, bsel[0:1, 0:1], bsel[1:2, 0:1])
        out_ref[...] = (stash_ref[pl.ds(j * RB, RB)] * g[:, :, None]
                        + b[:, :, None])


def kernel(x, gate_w, gate_b, gammas, betas):
    xs = x.reshape(RTOT, TH, LANES)
    out = pl.pallas_call(
        _body,
        grid=(2, NT),
        in_specs=[
            pl.BlockSpec((RB, TH, LANES),
                         lambda p, j: (jnp.where(p == 0, j, NT - 1), 0, 0)),
            pl.BlockSpec((NUM_EXPERTS, HID), lambda p, j: (0, 0)),
            pl.BlockSpec((NUM_EXPERTS, 1), lambda p, j: (0, 0)),
            pl.BlockSpec((NUM_EXPERTS, 1), lambda p, j: (0, 0)),
            pl.BlockSpec((NUM_EXPERTS, 1), lambda p, j: (0, 0)),
        ],
        out_specs=pl.BlockSpec(
            (RB, TH, LANES), lambda p, j: (jnp.where(p == 0, 0, j), 0, 0)),
        out_shape=jax.ShapeDtypeStruct((RTOT, TH, LANES), jnp.float32),
        scratch_shapes=[
            pltpu.VMEM((RTOT, TH, LANES), jnp.float32),
            pltpu.VMEM((B_ * NUM_EXPERTS, LANES), jnp.float32),
            pltpu.VMEM((B_, 1), jnp.float32),
            pltpu.VMEM((B_, 1), jnp.float32),
        ],
        compiler_params=pltpu.CompilerParams(
            dimension_semantics=("arbitrary", "arbitrary")),
    )(xs, gate_w, gate_b.reshape(NUM_EXPERTS, 1),
      gammas.reshape(NUM_EXPERTS, 1), betas.reshape(NUM_EXPERTS, 1))
    return out.reshape(B_, C_, T_, 1)
